# Initial kernel scaffold; baseline (speedup 1.0000x reference)
import jax
import jax.numpy as jnp
from jax.experimental import pallas as pl


def kernel(features_nchw, w0, b0, w1, b1, w2, b2, w3, b3, w4, b4, w5, b5, w6, b6, w7, b7, w8, b8):
    raise NotImplementedError("write your pallas kernel here")


# speedup vs baseline: 2.6292x; 2.6292x over previous
import jax
import jax.numpy as jnp
from jax import lax
from jax.experimental import pallas as pl
from jax.experimental.pallas import tpu as pltpu

_N = 32
_MARGIN = 16
_LEAD = 14


def _conv(a_ref, a_off, o_write, w_ref, b_ref, *, H, Wt, Cin, nchunks, relu,
          out_dtype):
    M = H * Wt
    mc = M // nchunks
    assert mc % 16 == 0
    bias = b_ref[...].astype(jnp.float32)

    def body(ci, carry):
        p0 = ci * mc
        acc = None
        for t in range(9):
            ky, kx = divmod(t, 3)
            off = a_off + ky * Wt + kx
            win = a_ref[pl.ds(p0 + (off // 16) * 16, mc + 16), :]
            a = win[off % 16:off % 16 + mc]
            w = w_ref[t * Cin:(t + 1) * Cin, :]
            d = jnp.dot(a, w, preferred_element_type=jnp.float32)
            acc = d if acc is None else acc + d
        acc = acc + bias
        if relu:
            acc = jnp.maximum(acc, 0.0)
        o_write(p0, acc.astype(out_dtype))
        return carry

    lax.fori_loop(0, nchunks, body, 0)


def _repad(o_ref, a_ref, *, H, Wt, nchunks):
    W = Wt - 2
    M = H * Wt
    mc = M // nchunks
    assert mc % 16 == 0 and (_LEAD + Wt) % 16 == 0

    def body(ci, carry):
        p0 = ci * mc
        jm = lax.rem(lax.broadcasted_iota(jnp.int32, (mc, 1), 0) + p0, Wt)
        win = o_ref[pl.ds(p0, mc + 32), :]
        s_p1 = win[_MARGIN + 1:_MARGIN + 1 + mc]
        s_m1 = win[_MARGIN - 1:_MARGIN - 1 + mc]
        s_m3 = win[_MARGIN - 3:_MARGIN - 3 + mc]
        mid = jnp.where(jm == 0, s_p1, jnp.where(jm <= W, s_m1, s_m3))
        a_ref[pl.ds(_LEAD + Wt + p0, mc), :] = mid
        return carry

    lax.fori_loop(0, nchunks, body, 0)
    a_ref[_LEAD:_LEAD + Wt, :] = a_ref[_LEAD + 2 * Wt:_LEAD + 3 * Wt, :]
    a_ref[_LEAD + (H + 1) * Wt:_LEAD + (H + 2) * Wt, :] = (
        a_ref[_LEAD + (H - 1) * Wt:_LEAD + H * Wt, :])


def _upsample_pad(o_ref, a_ref, *, Hs, Ws):
    wt_in = Ws + 2
    wt2 = 2 * Ws + 2
    G = 8
    assert Hs % G == 0 and (G * wt_in) % 16 == 0 and (_LEAD + wt2) % 16 == 0

    def selector(nrows_out, dup):
        t = lax.broadcasted_iota(jnp.int32, (nrows_out * wt2, G * wt_in), 0)
        s = lax.broadcasted_iota(jnp.int32, (nrows_out * wt2, G * wt_in), 1)
        d = lax.div(t, wt2)
        j = lax.rem(t, wt2)
        src = (lax.div(d, dup) * wt_in
               + jnp.clip(lax.div(j - 1, 2), 0, Ws - 1))
        return (s == src).astype(jnp.bfloat16)

    sel_dup = selector(2 * G, 2)
    sel_one = selector(1, 1)

    src_step = G * wt_in
    dst_base = _LEAD + wt2
    dst_step = 2 * G * wt2

    def body(g, carry):
        src = o_ref[pl.ds(_MARGIN + g * src_step, src_step), :]
        u = jnp.dot(sel_dup, src,
                    preferred_element_type=jnp.float32).astype(src.dtype)
        a_ref[pl.ds(dst_base + g * dst_step, dst_step), :] = u
        return carry

    lax.fori_loop(0, Hs // G, body, 0)
    src0 = o_ref[_MARGIN:_MARGIN + G * wt_in, :]
    srcZ = o_ref[_MARGIN + (Hs - 1) * wt_in:_MARGIN + (Hs - 1 + G) * wt_in, :]
    a_ref[_LEAD:_LEAD + wt2, :] = jnp.dot(
        sel_one, src0, preferred_element_type=jnp.float32).astype(src0.dtype)
    a_ref[_LEAD + (2 * Hs + 1) * wt2:_LEAD + (2 * Hs + 2) * wt2, :] = jnp.dot(
        sel_one, srcZ, preferred_element_type=jnp.float32).astype(srcZ.dtype)


def _decoder_body(x_ref,
                  w0, w1, w2, w3, w4, w5, w6, w7, w8,
                  b0, b1, b2, b3, b4, b5, b6, b7, b8,
                  o_ref, a256, o256, a128, o128, a64, o64):
    bf = jnp.bfloat16

    def wr(ref):
        return lambda p0, v: ref.__setitem__(
            (pl.ds(_MARGIN + p0, v.shape[0]), slice(None)), v)

    _conv(x_ref.at[0], 0, wr(o256), w0, b0,
          H=16, Wt=18, Cin=512, nchunks=1, relu=True, out_dtype=bf)
    _upsample_pad(o256, a256, Hs=16, Ws=16)
    _conv(a256, _LEAD, wr(o256), w1, b1, H=32, Wt=34, Cin=256, nchunks=4,
          relu=True, out_dtype=bf)
    _repad(o256, a256, H=32, Wt=34, nchunks=2)
    _conv(a256, _LEAD, wr(o256), w2, b2, H=32, Wt=34, Cin=256, nchunks=4,
          relu=True, out_dtype=bf)
    _repad(o256, a256, H=32, Wt=34, nchunks=2)
    _conv(a256, _LEAD, wr(o256), w3, b3, H=32, Wt=34, Cin=256, nchunks=4,
          relu=True, out_dtype=bf)
    _repad(o256, a256, H=32, Wt=34, nchunks=2)
    _conv(a256, _LEAD, wr(o128), w4, b4, H=32, Wt=34, Cin=256, nchunks=2,
          relu=True, out_dtype=bf)
    _upsample_pad(o128, a128, Hs=32, Ws=32)
    _conv(a128, _LEAD, wr(o128), w5, b5, H=64, Wt=66, Cin=128, nchunks=8,
          relu=True, out_dtype=bf)
    _repad(o128, a128, H=64, Wt=66, nchunks=4)
    _conv(a128, _LEAD, wr(o64), w6, b6, H=64, Wt=66, Cin=128, nchunks=4,
          relu=True, out_dtype=bf)
    _upsample_pad(o64, a64, Hs=64, Ws=64)
    _conv(a64, _LEAD, wr(o64), w7, b7, H=128, Wt=130, Cin=64, nchunks=16,
          relu=True, out_dtype=bf)
    _repad(o64, a64, H=128, Wt=130, nchunks=8)
    out2d = o_ref.at[0]
    _conv(a64, _LEAD,
          lambda p0, v: out2d.__setitem__(
              (pl.ds(p0, v.shape[0]), slice(None)), v),
          w8, b8, H=128, Wt=130, Cin=64, nchunks=4, relu=False,
          out_dtype=jnp.float32)


def kernel(features_nchw, w0, b0, w1, b1, w2, b2, w3, b3, w4, b4,
           w5, b5, w6, b6, w7, b7, w8, b8):
    x = jnp.transpose(features_nchw, (0, 2, 3, 1)).astype(jnp.bfloat16)
    x = jnp.pad(x, ((0, 0), (1, 1), (1, 1), (0, 0)), mode="reflect")
    x = x.reshape(_N, 18 * 18, 512)
    x = jnp.pad(x, ((0, 0), (0, 16), (0, 0)))

    cfg = [(512, 256), (256, 256), (256, 256), (256, 256), (256, 128),
           (128, 128), (128, 64), (64, 64), (64, 8)]
    ws, bs = [], []
    for (w, b), (cin, cpad) in zip(
            [(w0, b0), (w1, b1), (w2, b2), (w3, b3), (w4, b4),
             (w5, b5), (w6, b6), (w7, b7), (w8, b8)], cfg):
        cout = w.shape[-1]
        w9 = w.reshape(9 * cin, cout).astype(jnp.bfloat16)
        b2_ = b.reshape(1, cout).astype(jnp.float32)
        if cpad != cout:
            w9 = jnp.pad(w9, ((0, 0), (0, cpad - cout)))
            b2_ = jnp.pad(b2_, ((0, 0), (0, cpad - cout)))
        ws.append(w9)
        bs.append(b2_)

    full = lambda arr: pl.BlockSpec(arr.shape, lambda n: (0,) * arr.ndim)
    out_flat = pl.pallas_call(
        _decoder_body,
        grid=(_N,),
        in_specs=([pl.BlockSpec((1, 340, 512), lambda n: (n, 0, 0))]
                  + [full(w) for w in ws] + [full(b) for b in bs]),
        out_specs=pl.BlockSpec((1, 128 * 130, 8), lambda n: (n, 0, 0)),
        out_shape=jax.ShapeDtypeStruct((_N, 128 * 130, 8), jnp.float32),
        scratch_shapes=[
            pltpu.VMEM((1200, 256), jnp.bfloat16),
            pltpu.VMEM((32 * 34 + 32, 256), jnp.bfloat16),
            pltpu.VMEM((4400, 128), jnp.bfloat16),
            pltpu.VMEM((64 * 66 + 32, 128), jnp.bfloat16),
            pltpu.VMEM((16944, 64), jnp.bfloat16),
            pltpu.VMEM((128 * 130 + 32, 64), jnp.bfloat16),
        ],
        compiler_params=pltpu.CompilerParams(
            dimension_semantics=("parallel",),
            vmem_limit_bytes=64 * 1024 * 1024),
    )(x, *ws, *bs)

    out = out_flat.reshape(_N, 128, 130, 8)[:, :, :128, :3]
    return jnp.transpose(out, (0, 3, 1, 2))


# speedup vs baseline: 3.1594x; 1.2016x over previous
import jax
import jax.numpy as jnp
from jax import lax
from jax.experimental import pallas as pl
from jax.experimental.pallas import tpu as pltpu

_N = 32
_MARGIN = 16
_LEAD = 14


def _conv(a_ref, a_off, o_write, w_ref, b_ref, *, H, Wt, Cin, nchunks, relu,
          out_dtype):
    M = H * Wt
    mc = M // nchunks
    assert mc % 16 == 0
    bias = b_ref[...].astype(jnp.float32)

    kg = max(1, 256 // Cin)

    def tap(p0, t):
        ky, kx = divmod(t, 3)
        off = a_off + ky * Wt + kx
        win = a_ref[pl.ds(p0 + (off // 16) * 16, mc + 16), :]
        return win[off % 16:off % 16 + mc]

    def body(ci, carry):
        p0 = ci * mc
        acc = None
        for t0 in range(0, 9, kg):
            gs = min(kg, 9 - t0)
            a = (tap(p0, t0) if gs == 1 else jnp.concatenate(
                [tap(p0, t0 + i) for i in range(gs)], axis=1))
            w = w_ref[t0 * Cin:(t0 + gs) * Cin, :]
            d = jnp.dot(a, w, preferred_element_type=jnp.float32)
            acc = d if acc is None else acc + d
        acc = acc + bias
        if relu:
            acc = jnp.maximum(acc, 0.0)
        o_write(p0, acc.astype(out_dtype))
        return carry

    lax.fori_loop(0, nchunks, body, 0)


def _repad(o_ref, a_ref, *, H, Wt, nchunks):
    W = Wt - 2
    M = H * Wt
    mc = M // nchunks
    assert mc % 16 == 0 and (_LEAD + Wt) % 16 == 0

    def body(ci, carry):
        p0 = ci * mc
        jm = lax.rem(lax.broadcasted_iota(jnp.int32, (mc, 1), 0) + p0, Wt)
        win = o_ref[pl.ds(p0, mc + 32), :]
        s_p1 = win[_MARGIN + 1:_MARGIN + 1 + mc]
        s_m1 = win[_MARGIN - 1:_MARGIN - 1 + mc]
        s_m3 = win[_MARGIN - 3:_MARGIN - 3 + mc]
        mid = jnp.where(jm == 0, s_p1, jnp.where(jm <= W, s_m1, s_m3))
        a_ref[pl.ds(_LEAD + Wt + p0, mc), :] = mid
        return carry

    lax.fori_loop(0, nchunks, body, 0)
    a_ref[_LEAD:_LEAD + Wt, :] = a_ref[_LEAD + 2 * Wt:_LEAD + 3 * Wt, :]
    a_ref[_LEAD + (H + 1) * Wt:_LEAD + (H + 2) * Wt, :] = (
        a_ref[_LEAD + (H - 1) * Wt:_LEAD + H * Wt, :])


def _upsample_pad(o_ref, a_ref, *, Hs, Ws):
    wt_in = Ws + 2
    wt2 = 2 * Ws + 2
    G = 8
    assert Hs % G == 0 and (G * wt_in) % 16 == 0 and (_LEAD + wt2) % 16 == 0

    def selector(nrows_out, dup):
        t = lax.broadcasted_iota(jnp.int32, (nrows_out * wt2, G * wt_in), 0)
        s = lax.broadcasted_iota(jnp.int32, (nrows_out * wt2, G * wt_in), 1)
        d = lax.div(t, wt2)
        j = lax.rem(t, wt2)
        src = (lax.div(d, dup) * wt_in
               + jnp.clip(lax.div(j - 1, 2), 0, Ws - 1))
        return (s == src).astype(jnp.bfloat16)

    sel_dup = selector(2 * G, 2)
    sel_one = selector(1, 1)

    src_step = G * wt_in
    dst_base = _LEAD + wt2
    dst_step = 2 * G * wt2

    def body(g, carry):
        src = o_ref[pl.ds(_MARGIN + g * src_step, src_step), :]
        u = jnp.dot(sel_dup, src,
                    preferred_element_type=jnp.float32).astype(src.dtype)
        a_ref[pl.ds(dst_base + g * dst_step, dst_step), :] = u
        return carry

    lax.fori_loop(0, Hs // G, body, 0)
    src0 = o_ref[_MARGIN:_MARGIN + G * wt_in, :]
    srcZ = o_ref[_MARGIN + (Hs - 1) * wt_in:_MARGIN + (Hs - 1 + G) * wt_in, :]
    a_ref[_LEAD:_LEAD + wt2, :] = jnp.dot(
        sel_one, src0, preferred_element_type=jnp.float32).astype(src0.dtype)
    a_ref[_LEAD + (2 * Hs + 1) * wt2:_LEAD + (2 * Hs + 2) * wt2, :] = jnp.dot(
        sel_one, srcZ, preferred_element_type=jnp.float32).astype(srcZ.dtype)


def _decoder_body(x_ref,
                  w0, w1, w2, w3, w4, w5, w6, w7, w8,
                  b0, b1, b2, b3, b4, b5, b6, b7, b8,
                  o_ref, a256, o256, a128, o128, a64, o64):
    bf = jnp.bfloat16

    def wr(ref):
        return lambda p0, v: ref.__setitem__(
            (pl.ds(_MARGIN + p0, v.shape[0]), slice(None)), v)

    _conv(x_ref.at[0], 0, wr(o256), w0, b0,
          H=16, Wt=18, Cin=512, nchunks=1, relu=True, out_dtype=bf)
    _upsample_pad(o256, a256, Hs=16, Ws=16)
    _conv(a256, _LEAD, wr(o256), w1, b1, H=32, Wt=34, Cin=256, nchunks=4,
          relu=True, out_dtype=bf)
    _repad(o256, a256, H=32, Wt=34, nchunks=2)
    _conv(a256, _LEAD, wr(o256), w2, b2, H=32, Wt=34, Cin=256, nchunks=4,
          relu=True, out_dtype=bf)
    _repad(o256, a256, H=32, Wt=34, nchunks=2)
    _conv(a256, _LEAD, wr(o256), w3, b3, H=32, Wt=34, Cin=256, nchunks=4,
          relu=True, out_dtype=bf)
    _repad(o256, a256, H=32, Wt=34, nchunks=2)
    _conv(a256, _LEAD, wr(o128), w4, b4, H=32, Wt=34, Cin=256, nchunks=2,
          relu=True, out_dtype=bf)
    _upsample_pad(o128, a128, Hs=32, Ws=32)
    _conv(a128, _LEAD, wr(o128), w5, b5, H=64, Wt=66, Cin=128, nchunks=8,
          relu=True, out_dtype=bf)
    _repad(o128, a128, H=64, Wt=66, nchunks=4)
    _conv(a128, _LEAD, wr(o64), w6, b6, H=64, Wt=66, Cin=128, nchunks=4,
          relu=True, out_dtype=bf)
    _upsample_pad(o64, a64, Hs=64, Ws=64)
    _conv(a64, _LEAD, wr(o64), w7, b7, H=128, Wt=130, Cin=64, nchunks=16,
          relu=True, out_dtype=bf)
    _repad(o64, a64, H=128, Wt=130, nchunks=8)
    out2d = o_ref.at[0]
    _conv(a64, _LEAD,
          lambda p0, v: out2d.__setitem__(
              (pl.ds(p0, v.shape[0]), slice(None)), v),
          w8, b8, H=128, Wt=130, Cin=64, nchunks=4, relu=False,
          out_dtype=jnp.float32)


def kernel(features_nchw, w0, b0, w1, b1, w2, b2, w3, b3, w4, b4,
           w5, b5, w6, b6, w7, b7, w8, b8):
    x = jnp.transpose(features_nchw, (0, 2, 3, 1)).astype(jnp.bfloat16)
    x = jnp.pad(x, ((0, 0), (1, 1), (1, 1), (0, 0)), mode="reflect")
    x = x.reshape(_N, 18 * 18, 512)
    x = jnp.pad(x, ((0, 0), (0, 16), (0, 0)))

    cfg = [(512, 256), (256, 256), (256, 256), (256, 256), (256, 128),
           (128, 128), (128, 64), (64, 64), (64, 8)]
    ws, bs = [], []
    for (w, b), (cin, cpad) in zip(
            [(w0, b0), (w1, b1), (w2, b2), (w3, b3), (w4, b4),
             (w5, b5), (w6, b6), (w7, b7), (w8, b8)], cfg):
        cout = w.shape[-1]
        w9 = w.reshape(9 * cin, cout).astype(jnp.bfloat16)
        b2_ = b.reshape(1, cout).astype(jnp.float32)
        if cpad != cout:
            w9 = jnp.pad(w9, ((0, 0), (0, cpad - cout)))
            b2_ = jnp.pad(b2_, ((0, 0), (0, cpad - cout)))
        ws.append(w9)
        bs.append(b2_)

    full = lambda arr: pl.BlockSpec(arr.shape, lambda n: (0,) * arr.ndim)
    out_flat = pl.pallas_call(
        _decoder_body,
        grid=(_N,),
        in_specs=([pl.BlockSpec((1, 340, 512), lambda n: (n, 0, 0))]
                  + [full(w) for w in ws] + [full(b) for b in bs]),
        out_specs=pl.BlockSpec((1, 128 * 130, 8), lambda n: (n, 0, 0)),
        out_shape=jax.ShapeDtypeStruct((_N, 128 * 130, 8), jnp.float32),
        scratch_shapes=[
            pltpu.VMEM((1200, 256), jnp.bfloat16),
            pltpu.VMEM((32 * 34 + 32, 256), jnp.bfloat16),
            pltpu.VMEM((4400, 128), jnp.bfloat16),
            pltpu.VMEM((64 * 66 + 32, 128), jnp.bfloat16),
            pltpu.VMEM((16944, 64), jnp.bfloat16),
            pltpu.VMEM((128 * 130 + 32, 64), jnp.bfloat16),
        ],
        compiler_params=pltpu.CompilerParams(
            dimension_semantics=("parallel",),
            vmem_limit_bytes=64 * 1024 * 1024),
    )(x, *ws, *bs)

    out = out_flat.reshape(_N, 128, 130, 8)[:, :, :128, :3]
    return jnp.transpose(out, (0, 3, 1, 2))


# speedup vs baseline: 3.2186x; 1.0187x over previous
import jax
import jax.numpy as jnp
from jax import lax
from jax.experimental import pallas as pl
from jax.experimental.pallas import tpu as pltpu

_N = 32
_MARGIN = 16
_LEAD = 14


def _conv(a_ref, a_off, o_write, w_ref, b_ref, *, H, Wt, Cin, nchunks, relu,
          out_dtype):
    M = H * Wt
    mc = M // nchunks
    assert mc % 16 == 0
    bias = b_ref[...].astype(jnp.float32)

    kg = max(1, 256 // Cin)

    def tap(p0, t):
        ky, kx = divmod(t, 3)
        off = a_off + ky * Wt + kx
        win = a_ref[pl.ds(p0 + (off // 16) * 16, mc + 16), :]
        return win[off % 16:off % 16 + mc]

    def body(ci, carry):
        p0 = ci * mc
        acc = None
        for t0 in range(0, 9, kg):
            gs = min(kg, 9 - t0)
            a = (tap(p0, t0) if gs == 1 else jnp.concatenate(
                [tap(p0, t0 + i) for i in range(gs)], axis=1))
            w = w_ref[t0 * Cin:(t0 + gs) * Cin, :]
            d = jnp.dot(a, w, preferred_element_type=jnp.float32)
            acc = d if acc is None else acc + d
        acc = acc + bias
        if relu:
            acc = jnp.maximum(acc, 0.0)
        o_write(p0, acc.astype(out_dtype))
        return carry

    lax.fori_loop(0, nchunks, body, 0)


def _repad(o_ref, a_ref, *, H, Wt, nchunks):
    W = Wt - 2
    M = H * Wt
    mc = M // nchunks
    assert mc % 16 == 0 and (_LEAD + Wt) % 16 == 0

    def body(ci, carry):
        p0 = ci * mc
        jm = lax.rem(lax.broadcasted_iota(jnp.int32, (mc, 1), 0) + p0, Wt)
        win = o_ref[pl.ds(p0, mc + 32), :]
        s_p1 = win[_MARGIN + 1:_MARGIN + 1 + mc]
        s_m1 = win[_MARGIN - 1:_MARGIN - 1 + mc]
        s_m3 = win[_MARGIN - 3:_MARGIN - 3 + mc]
        mid = jnp.where(jm == 0, s_p1, jnp.where(jm <= W, s_m1, s_m3))
        a_ref[pl.ds(_LEAD + Wt + p0, mc), :] = mid
        return carry

    lax.fori_loop(0, nchunks, body, 0)
    a_ref[_LEAD:_LEAD + Wt, :] = a_ref[_LEAD + 2 * Wt:_LEAD + 3 * Wt, :]
    a_ref[_LEAD + (H + 1) * Wt:_LEAD + (H + 2) * Wt, :] = (
        a_ref[_LEAD + (H - 1) * Wt:_LEAD + H * Wt, :])


def _upsample_pad(o_ref, a_ref, *, Hs, Ws):
    wt_in = Ws + 2
    wt2 = 2 * Ws + 2
    G = 8
    assert Hs % G == 0 and (G * wt_in) % 16 == 0 and (_LEAD + wt2) % 16 == 0

    def selector(nrows_out, dup):
        t = lax.broadcasted_iota(jnp.int32, (nrows_out * wt2, G * wt_in), 0)
        s = lax.broadcasted_iota(jnp.int32, (nrows_out * wt2, G * wt_in), 1)
        d = lax.div(t, wt2)
        j = lax.rem(t, wt2)
        src = (lax.div(d, dup) * wt_in
               + jnp.clip(lax.div(j - 1, 2), 0, Ws - 1))
        return (s == src).astype(jnp.bfloat16)

    sel_dup = selector(2 * G, 2)
    sel_one = selector(1, 1)
    half_k = G // 2 * wt_in
    sel_half = sel_dup[:G * wt2, :half_k]

    src_step = G * wt_in
    dst_base = _LEAD + wt2
    dst_step = 2 * G * wt2

    def body(g, carry):
        src = o_ref[pl.ds(_MARGIN + g * src_step, src_step), :]
        for i in range(2):
            u = jnp.dot(sel_half, src[i * half_k:(i + 1) * half_k],
                        preferred_element_type=jnp.float32).astype(src.dtype)
            a_ref[pl.ds(dst_base + g * dst_step + i * (G * wt2),
                        G * wt2), :] = u
        return carry

    lax.fori_loop(0, Hs // G, body, 0)
    src0 = o_ref[_MARGIN:_MARGIN + G * wt_in, :]
    srcZ = o_ref[_MARGIN + (Hs - 1) * wt_in:_MARGIN + (Hs - 1 + G) * wt_in, :]
    a_ref[_LEAD:_LEAD + wt2, :] = jnp.dot(
        sel_one, src0, preferred_element_type=jnp.float32).astype(src0.dtype)
    a_ref[_LEAD + (2 * Hs + 1) * wt2:_LEAD + (2 * Hs + 2) * wt2, :] = jnp.dot(
        sel_one, srcZ, preferred_element_type=jnp.float32).astype(srcZ.dtype)


def _decoder_body(x_ref,
                  w0, w1, w2, w3, w4, w5, w6, w7, w8,
                  b0, b1, b2, b3, b4, b5, b6, b7, b8,
                  o_ref, a256, o256, a128, o128, a64, o64):
    bf = jnp.bfloat16

    def wr(ref):
        return lambda p0, v: ref.__setitem__(
            (pl.ds(_MARGIN + p0, v.shape[0]), slice(None)), v)

    _conv(x_ref.at[0], 0, wr(o256), w0, b0,
          H=16, Wt=18, Cin=512, nchunks=1, relu=True, out_dtype=bf)
    _upsample_pad(o256, a256, Hs=16, Ws=16)
    _conv(a256, _LEAD, wr(o256), w1, b1, H=32, Wt=34, Cin=256, nchunks=4,
          relu=True, out_dtype=bf)
    _repad(o256, a256, H=32, Wt=34, nchunks=2)
    _conv(a256, _LEAD, wr(o256), w2, b2, H=32, Wt=34, Cin=256, nchunks=4,
          relu=True, out_dtype=bf)
    _repad(o256, a256, H=32, Wt=34, nchunks=2)
    _conv(a256, _LEAD, wr(o256), w3, b3, H=32, Wt=34, Cin=256, nchunks=4,
          relu=True, out_dtype=bf)
    _repad(o256, a256, H=32, Wt=34, nchunks=2)
    _conv(a256, _LEAD, wr(o128), w4, b4, H=32, Wt=34, Cin=256, nchunks=2,
          relu=True, out_dtype=bf)
    _upsample_pad(o128, a128, Hs=32, Ws=32)
    _conv(a128, _LEAD, wr(o128), w5, b5, H=64, Wt=66, Cin=128, nchunks=8,
          relu=True, out_dtype=bf)
    _repad(o128, a128, H=64, Wt=66, nchunks=4)
    _conv(a128, _LEAD, wr(o64), w6, b6, H=64, Wt=66, Cin=128, nchunks=4,
          relu=True, out_dtype=bf)
    _upsample_pad(o64, a64, Hs=64, Ws=64)
    _conv(a64, _LEAD, wr(o64), w7, b7, H=128, Wt=130, Cin=64, nchunks=16,
          relu=True, out_dtype=bf)
    _repad(o64, a64, H=128, Wt=130, nchunks=8)
    out2d = o_ref.at[0]
    _conv(a64, _LEAD,
          lambda p0, v: out2d.__setitem__(
              (pl.ds(p0, v.shape[0]), slice(None)), v),
          w8, b8, H=128, Wt=130, Cin=64, nchunks=4, relu=False,
          out_dtype=jnp.float32)


def kernel(features_nchw, w0, b0, w1, b1, w2, b2, w3, b3, w4, b4,
           w5, b5, w6, b6, w7, b7, w8, b8):
    x = jnp.transpose(features_nchw, (0, 2, 3, 1)).astype(jnp.bfloat16)
    x = jnp.pad(x, ((0, 0), (1, 1), (1, 1), (0, 0)), mode="reflect")
    x = x.reshape(_N, 18 * 18, 512)
    x = jnp.pad(x, ((0, 0), (0, 16), (0, 0)))

    cfg = [(512, 256), (256, 256), (256, 256), (256, 256), (256, 128),
           (128, 128), (128, 64), (64, 64), (64, 8)]
    ws, bs = [], []
    for (w, b), (cin, cpad) in zip(
            [(w0, b0), (w1, b1), (w2, b2), (w3, b3), (w4, b4),
             (w5, b5), (w6, b6), (w7, b7), (w8, b8)], cfg):
        cout = w.shape[-1]
        w9 = w.reshape(9 * cin, cout).astype(jnp.bfloat16)
        b2_ = b.reshape(1, cout).astype(jnp.float32)
        if cpad != cout:
            w9 = jnp.pad(w9, ((0, 0), (0, cpad - cout)))
            b2_ = jnp.pad(b2_, ((0, 0), (0, cpad - cout)))
        ws.append(w9)
        bs.append(b2_)

    full = lambda arr: pl.BlockSpec(arr.shape, lambda n: (0,) * arr.ndim)
    out_flat = pl.pallas_call(
        _decoder_body,
        grid=(_N,),
        in_specs=([pl.BlockSpec((1, 340, 512), lambda n: (n, 0, 0))]
                  + [full(w) for w in ws] + [full(b) for b in bs]),
        out_specs=pl.BlockSpec((1, 128 * 130, 8), lambda n: (n, 0, 0)),
        out_shape=jax.ShapeDtypeStruct((_N, 128 * 130, 8), jnp.float32),
        scratch_shapes=[
            pltpu.VMEM((1200, 256), jnp.bfloat16),
            pltpu.VMEM((32 * 34 + 32, 256), jnp.bfloat16),
            pltpu.VMEM((4400, 128), jnp.bfloat16),
            pltpu.VMEM((64 * 66 + 32, 128), jnp.bfloat16),
            pltpu.VMEM((16944, 64), jnp.bfloat16),
            pltpu.VMEM((128 * 130 + 32, 64), jnp.bfloat16),
        ],
        compiler_params=pltpu.CompilerParams(
            dimension_semantics=("parallel",),
            vmem_limit_bytes=64 * 1024 * 1024),
    )(x, *ws, *bs)

    out = out_flat.reshape(_N, 128, 130, 8)[:, :, :128, :3]
    return jnp.transpose(out, (0, 3, 1, 2))
